# baseline (speedup 1.0000x reference)
import dataclasses

import jax
import jax.numpy as jnp
from jax import lax
from jax.experimental import pallas as pl
from jax.experimental.pallas import tpu as pltpu
from jax.experimental.pallas import tpu_sc as plsc

_USER_NUM = 1000000
_TABLE_ROWS = 2 * _USER_NUM
_B = 16384
_F = 16
_NC = 2
_NS = 16
_NW = _NC * _NS
_BPW = _B // _NW
_LANES = 16
_NSL = _BPW // _LANES
_CHUNK = 128
_NCH = _BPW // _CHUNK
_FBASE = [(f // 8) * (_TABLE_ROWS * 8) + (f % 8) * 128 for f in range(_F)]


def _fm_sc_body(user_ref, item_ref, emb_ref, lin_ref, bias_ref, out_ref,
                uidx, iidx, uidxb, iidxb, ubufT, ibufT, ulin, ilin,
                outv, biasv, sem, lsem):
    wid = lax.axis_index("s") * _NC + lax.axis_index("c")
    base = wid * _BPW

    pltpu.sync_copy(user_ref.at[pl.ds(base, _BPW)], uidx)
    pltpu.sync_copy(item_ref.at[pl.ds(base, _BPW)], iidx)
    pltpu.sync_copy(bias_ref, biasv)

    @pl.loop(0, _NSL)
    def _(s):
        sl = pl.ds(s * _LANES, _LANES)
        uv = uidx[sl]
        iv = iidx[sl] + _USER_NUM
        iidx[sl] = iv
        ub = ((uv >> 7) << 10) + (uv & 127)
        ib = ((iv >> 7) << 10) + (iv & 127)
        for f in range(_F):
            bsl = pl.ds(f * _BPW + s * _LANES, _LANES)
            uidxb[bsl] = ub + _FBASE[f]
            iidxb[bsl] = ib + _FBASE[f]

    cps = (pltpu.async_copy(emb_ref.at[uidxb], ubufT, sem),
           pltpu.async_copy(emb_ref.at[iidxb], ibufT, sem),
           pltpu.async_copy(lin_ref.at[0].at[uidx], ulin, lsem),
           pltpu.async_copy(lin_ref.at[0].at[iidx], ilin, lsem))
    for cp in cps:
        cp.wait()

    b = biasv[...]

    @pl.loop(0, _NSL)
    def _(s):
        sl = pl.ds(s * _LANES, _LANES)
        acc = ulin[sl] + ilin[sl] + b
        for f in range(_F):
            fsl = pl.ds(f * _BPW + s * _LANES, _LANES)
            acc = acc + ubufT[fsl] * ibufT[fsl]
        outv[sl] = acc

    pltpu.sync_copy(outv, out_ref.at[pl.ds(base, _BPW)])


def kernel(user, item, emb_table, lin_table, bias):
    emb_flat = (emb_table
                .reshape(_TABLE_ROWS // 128, 128, 2, 8)
                .transpose(2, 0, 3, 1)
                .reshape(_TABLE_ROWS * _F))
    bias16 = jnp.broadcast_to(bias, (_LANES,))
    mesh = plsc.VectorSubcoreMesh(core_axis_name="c", subcore_axis_name="s")
    cp = pltpu.CompilerParams()
    for fld, val in (("needs_layout_passes", False),
                     ("use_tc_tiling_on_sc", False)):
        if fld in pltpu.CompilerParams.__dataclass_fields__:
            cp = dataclasses.replace(cp, **{fld: val})
    f = pl.kernel(
        _fm_sc_body,
        out_type=jax.ShapeDtypeStruct((_B,), jnp.float32),
        mesh=mesh,
        scratch_types=[
            pltpu.VMEM((_BPW,), jnp.int32),
            pltpu.VMEM((_BPW,), jnp.int32),
            pltpu.VMEM((_F * _BPW,), jnp.int32),
            pltpu.VMEM((_F * _BPW,), jnp.int32),
            pltpu.VMEM((_F * _BPW,), jnp.float32),
            pltpu.VMEM((_F * _BPW,), jnp.float32),
            pltpu.VMEM((_BPW,), jnp.float32),
            pltpu.VMEM((_BPW,), jnp.float32),
            pltpu.VMEM((_BPW,), jnp.float32),
            pltpu.VMEM((_LANES,), jnp.float32),
            pltpu.SemaphoreType.DMA,
            pltpu.SemaphoreType.DMA,
        ],
        compiler_params=cp,
    )
    return f(user, item, emb_flat, lin_table.T, bias16)


# speedup vs baseline: 1.2708x; 1.2708x over previous
import dataclasses

import jax
import jax.numpy as jnp
from jax import lax
from jax.experimental import pallas as pl
from jax.experimental.pallas import tpu as pltpu
from jax.experimental.pallas import tpu_sc as plsc

_USER_NUM = 1000000
_TABLE_ROWS = 2 * _USER_NUM
_B = 16384
_F = 16
_NC = 2
_NS = 16
_NW = _NC * _NS
_BPW = _B // _NW
_LANES = 16
_NSL = _BPW // _LANES
_FBASE = [(f // 8) * (_TABLE_ROWS * 8) + (f % 8) * 128 for f in range(_F)]


def _emb_sc_body(user_ref, item_ref, emb_ref, bias_ref, out_ref,
                 uidx, iidx, uidxb, iidxb, ubufT, ibufT, outv, biasv, sem):
    wid = lax.axis_index("s") * _NC + lax.axis_index("c")
    base = wid * _BPW

    pltpu.sync_copy(user_ref.at[pl.ds(base, _BPW)], uidx)
    pltpu.sync_copy(item_ref.at[pl.ds(base, _BPW)], iidx)
    pltpu.sync_copy(bias_ref, biasv)

    @pl.loop(0, _NSL)
    def _(s):
        sl = pl.ds(s * _LANES, _LANES)
        uv = uidx[sl]
        iv = iidx[sl] + _USER_NUM
        ub = ((uv >> 7) << 10) + (uv & 127)
        ib = ((iv >> 7) << 10) + (iv & 127)
        for f in range(_F):
            bsl = pl.ds(f * _BPW + s * _LANES, _LANES)
            uidxb[bsl] = ub + _FBASE[f]
            iidxb[bsl] = ib + _FBASE[f]

    cps = (pltpu.async_copy(emb_ref.at[uidxb], ubufT, sem),
           pltpu.async_copy(emb_ref.at[iidxb], ibufT, sem))
    for cp in cps:
        cp.wait()

    b = biasv[...]

    @pl.loop(0, _NSL)
    def _(s):
        sl = pl.ds(s * _LANES, _LANES)
        acc = b
        for f in range(_F):
            fsl = pl.ds(f * _BPW + s * _LANES, _LANES)
            acc = acc + ubufT[fsl] * ibufT[fsl]
        outv[sl] = acc

    pltpu.sync_copy(outv, out_ref.at[pl.ds(base, _BPW)])


def _lin_sc_body(user_ref, item_ref, lin_ref, part_ref, out_ref,
                 uidx, iidx, ulin, ilin, outv, sem):
    wid = lax.axis_index("s") * _NC + lax.axis_index("c")
    base = wid * _BPW

    pltpu.sync_copy(user_ref.at[pl.ds(base, _BPW)], uidx)
    pltpu.sync_copy(item_ref.at[pl.ds(base, _BPW)], iidx)
    pltpu.sync_copy(part_ref.at[pl.ds(base, _BPW)], outv)

    @pl.loop(0, _NSL)
    def _(s):
        sl = pl.ds(s * _LANES, _LANES)
        iidx[sl] = iidx[sl] + _USER_NUM

    cps = (pltpu.async_copy(lin_ref.at[uidx], ulin, sem),
           pltpu.async_copy(lin_ref.at[iidx], ilin, sem))
    for cp in cps:
        cp.wait()

    @pl.loop(0, _NSL)
    def _(s):
        sl = pl.ds(s * _LANES, _LANES)
        outv[sl] = outv[sl] + ulin[sl] + ilin[sl]

    pltpu.sync_copy(outv, out_ref.at[pl.ds(base, _BPW)])


def _compiler_params():
    cp = pltpu.CompilerParams()
    for fld, val in (("needs_layout_passes", False),
                     ("use_tc_tiling_on_sc", False)):
        if fld in pltpu.CompilerParams.__dataclass_fields__:
            cp = dataclasses.replace(cp, **{fld: val})
    return cp


def kernel(user, item, emb_table, lin_table, bias):
    emb_flat = (emb_table
                .reshape(_TABLE_ROWS // 128, 128, 2, 8)
                .transpose(2, 0, 3, 1)
                .reshape(_TABLE_ROWS * _F))
    lin_flat = lin_table.reshape(_TABLE_ROWS)
    bias16 = jnp.broadcast_to(bias, (_LANES,))
    mesh = plsc.VectorSubcoreMesh(core_axis_name="c", subcore_axis_name="s")
    cp = _compiler_params()
    fa = pl.kernel(
        _emb_sc_body,
        out_type=jax.ShapeDtypeStruct((_B,), jnp.float32),
        mesh=mesh,
        scratch_types=[
            pltpu.VMEM((_BPW,), jnp.int32),
            pltpu.VMEM((_BPW,), jnp.int32),
            pltpu.VMEM((_F * _BPW,), jnp.int32),
            pltpu.VMEM((_F * _BPW,), jnp.int32),
            pltpu.VMEM((_F * _BPW,), jnp.float32),
            pltpu.VMEM((_F * _BPW,), jnp.float32),
            pltpu.VMEM((_BPW,), jnp.float32),
            pltpu.VMEM((_LANES,), jnp.float32),
            pltpu.SemaphoreType.DMA,
        ],
        compiler_params=cp,
    )
    fb = pl.kernel(
        _lin_sc_body,
        out_type=jax.ShapeDtypeStruct((_B,), jnp.float32),
        mesh=mesh,
        scratch_types=[
            pltpu.VMEM((_BPW,), jnp.int32),
            pltpu.VMEM((_BPW,), jnp.int32),
            pltpu.VMEM((_BPW,), jnp.float32),
            pltpu.VMEM((_BPW,), jnp.float32),
            pltpu.VMEM((_BPW,), jnp.float32),
            pltpu.SemaphoreType.DMA,
        ],
        compiler_params=cp,
    )
    partial = fa(user, item, emb_flat, bias16)
    return fb(user, item, lin_flat, partial)
